# baseline (speedup 1.0000x reference)
import functools

import jax
import jax.numpy as jnp
from jax.experimental import pallas as pl
from jax.experimental.pallas import tpu as pltpu


_VMEM_LIMIT = 48 * 1024 * 1024


def _extend(t, axis, n):
    shp = list(t.shape)
    shp[axis] = n
    return jnp.concatenate([t, jnp.zeros(shp, t.dtype)], axis=axis)


def _zpad2(t, p):
    tb, h, w, c = t.shape
    zr = jnp.zeros((tb, p, w, c), t.dtype)
    t = jnp.concatenate([zr, t, zr], axis=1)
    zc = jnp.zeros((tb, h + 2 * p, p, c), t.dtype)
    return jnp.concatenate([zc, t, zc], axis=2)


def _pool3x2(t, oh, ow):
    tb, h, w, c = t.shape
    r = t.reshape(tb, h // 2, 2, w, c)
    t = jnp.maximum(jnp.maximum(r[:, :oh, 0], r[:, :oh, 1]), r[:, 1:oh + 1, 0])
    r = t.reshape(tb, oh, w // 2, 2, c)
    return jnp.maximum(jnp.maximum(r[:, :, :ow, 0], r[:, :, :ow, 1]),
                       r[:, :, 1:ow + 1, 0])


def _c0_body(x_ref, w_ref, b_ref, o_ref):
    tb = x_ref.shape[0]
    v = jnp.swapaxes(x_ref[...], 2, 3)
    v = v.reshape(tb, 57, 57, 4, 12)
    xs = jnp.concatenate([v[:, :, :, jm, :] for jm in range(4)],
                         axis=-1)
    x = _extend(_extend(xs, 1, 2), 2, 2)
    a = jnp.concatenate(
        [x[:, gi:gi + 56, gj:gj + 56, :]
         for gi in range(3) for gj in range(3)], axis=-1)
    a = a.reshape(tb * 56 * 56, 9 * 48)
    acc = jnp.dot(a, w_ref[...], preferred_element_type=jnp.float32)
    y = jnp.maximum(acc + b_ref[...], 0.0)
    y = y.reshape(tb, 56, 56, 48)
    y = _pool3x2(y, 27, 27)
    o_ref[...] = _zpad2(y, 2).astype(o_ref.dtype)


def _c1_body(x_ref, w_ref, b_ref, o_ref):
    tb = x_ref.shape[0]
    x = _extend(_extend(x_ref[...], 1, 5), 2, 5)
    acc = None
    for i in range(5):
        a = jnp.concatenate(
            [x[:, i:i + 32, j:j + 32, :] for j in range(5)], axis=-1)
        a = a.reshape(tb * 32 * 32, 240)
        d = jnp.dot(a, w_ref[i], preferred_element_type=jnp.float32)
        acc = d if acc is None else acc + d
    y = jnp.maximum(acc + b_ref[...], 0.0)
    y = y.reshape(tb, 32, 32, 128)
    y = _pool3x2(y, 13, 13)
    o_ref[...] = _zpad2(y, 1).astype(o_ref.dtype)


def _conv3x3(x, w, b, n):
    tb, _, _, c = x.shape
    xe = _extend(_extend(x, 1, 3), 2, 3)
    a = jnp.concatenate(
        [xe[:, i:i + 16, j:j + 16, :]
         for i in range(3) for j in range(3)], axis=-1)
    a = a.reshape(tb * 16 * 16, 9 * c)
    y = jnp.dot(a, w, preferred_element_type=jnp.float32)
    y = jnp.maximum(y + b, 0.0)
    return y.reshape(tb, 16, 16, n)


def _c234_body(x_ref, w2_ref, b2_ref, w3_ref, b3_ref, w4_ref, b4_ref, o_ref):
    y = _conv3x3(x_ref[...], w2_ref[...], b2_ref[...], 192)
    y = _zpad2(y[:, :13, :13, :].astype(jnp.bfloat16), 1)
    y = _conv3x3(y, w3_ref[...], b3_ref[...], 192)
    y = _zpad2(y[:, :13, :13, :].astype(jnp.bfloat16), 1)
    y = _conv3x3(y, w4_ref[...], b4_ref[...], 128)
    o_ref[...] = _pool3x2(y, 6, 6).astype(o_ref.dtype)


def _conv_call(body, x, wbs, out_shape, out_dtype, tb, vmem=_VMEM_LIMIT):
    batch = x.shape[0]
    xs = x.shape[1:]
    os = out_shape
    wb_specs = [pl.BlockSpec(a.shape, functools.partial(
        lambda nd, i: (0,) * nd, a.ndim)) for a in wbs]
    return pl.pallas_call(
        body,
        out_shape=jax.ShapeDtypeStruct((batch,) + os, out_dtype),
        grid=(batch // tb,),
        in_specs=[pl.BlockSpec((tb,) + xs, lambda i: (i, 0, 0, 0))] + wb_specs,
        out_specs=pl.BlockSpec((tb,) + os, lambda i: (i, 0, 0, 0)),
        compiler_params=pltpu.CompilerParams(
            dimension_semantics=("parallel",),
            vmem_limit_bytes=vmem),
    )(x, *wbs)


def _fc_body(x_ref, w1_ref, b1_ref, w2_ref, b2_ref, w3_ref, b3_ref, o_ref):
    h = jnp.dot(x_ref[...], w1_ref[...], preferred_element_type=jnp.float32)
    h = jnp.maximum(h + b1_ref[...], 0.0).astype(jnp.bfloat16)
    h = jnp.dot(h, w2_ref[...], preferred_element_type=jnp.float32)
    h = jnp.maximum(h + b2_ref[...], 0.0).astype(jnp.bfloat16)
    o = jnp.dot(h, w3_ref[...], preferred_element_type=jnp.float32)
    o_ref[...] = o + b3_ref[...]


def _classifier(xf, w1, b1, w2, b2, w3, b3):
    batch, k1 = xf.shape
    nb = 2 if batch % 2 == 0 else 1
    tb = batch // nb
    nc = w3.shape[1]
    return pl.pallas_call(
        _fc_body,
        out_shape=jax.ShapeDtypeStruct((batch, nc), jnp.float32),
        grid=(nb,),
        in_specs=[
            pl.BlockSpec((tb, k1), lambda i: (i, 0)),
            pl.BlockSpec(w1.shape, lambda i: (0, 0)),
            pl.BlockSpec(b1.shape, lambda i: (0, 0)),
            pl.BlockSpec(w2.shape, lambda i: (0, 0)),
            pl.BlockSpec(b2.shape, lambda i: (0, 0)),
            pl.BlockSpec(w3.shape, lambda i: (0, 0)),
            pl.BlockSpec(b3.shape, lambda i: (0, 0)),
        ],
        out_specs=pl.BlockSpec((tb, nc), lambda i: (i, 0)),
        compiler_params=pltpu.CompilerParams(
            dimension_semantics=("parallel",),
            vmem_limit_bytes=_VMEM_LIMIT),
    )(xf, w1, b1, w2, b2, w3, b3)


def kernel(x, conv0_w, conv0_b, conv1_w, conv1_b, conv2_w, conv2_b,
           conv3_w, conv3_b, conv4_w, conv4_b,
           fc0_w, fc0_b, fc1_w, fc1_b, fc2_w, fc2_b):
    batch = x.shape[0]

    xb = x.astype(jnp.bfloat16)
    xp = jnp.pad(xb, ((0, 0), (0, 0), (2, 2), (2, 2)))
    xs = xp.transpose(0, 2, 1, 3)
    xs = xs.reshape(batch, 57, 12, 228)

    w4 = conv0_w.reshape(11, 11, 3, 48)
    w4 = jnp.pad(w4, ((0, 1), (0, 1), (0, 0), (0, 0)))
    w0 = w4.reshape(3, 4, 3, 4, 3, 48).transpose(0, 2, 3, 1, 4, 5)
    w0 = w0.reshape(432, 48)

    y = _conv_call(_c0_body, xs, [w0, conv0_b], (31, 31, 48),
                   jnp.bfloat16, tb=2, vmem=60 * 1024 * 1024)
    y = _conv_call(_c1_body, y, [conv1_w.reshape(5, 240, 128), conv1_b],
                   (15, 15, 128), jnp.bfloat16, tb=8)
    y = _conv_call(_c234_body, y,
                   [conv2_w, conv2_b, conv3_w, conv3_b, conv4_w, conv4_b],
                   (6, 6, 128), jnp.bfloat16, tb=8)

    xf = y.reshape(batch, 6 * 6 * 128)
    return _classifier(xf, fc0_w, fc0_b, fc1_w, fc1_b, fc2_w, fc2_b)


# speedup vs baseline: 1.3135x; 1.3135x over previous
import functools

import jax
import jax.numpy as jnp
from jax.experimental import pallas as pl
from jax.experimental.pallas import tpu as pltpu


_VMEM_LIMIT = 48 * 1024 * 1024


def _extend(t, axis, n):
    shp = list(t.shape)
    shp[axis] = n
    return jnp.concatenate([t, jnp.zeros(shp, t.dtype)], axis=axis)


def _zpad2(t, p):
    tb, h, w, c = t.shape
    zr = jnp.zeros((tb, p, w, c), t.dtype)
    t = jnp.concatenate([zr, t, zr], axis=1)
    zc = jnp.zeros((tb, h + 2 * p, p, c), t.dtype)
    return jnp.concatenate([zc, t, zc], axis=2)


def _pool3x2(t, oh, ow):
    tb, h, w, c = t.shape
    r = t.reshape(tb, h // 2, 2, w, c)
    t = jnp.maximum(jnp.maximum(r[:, :oh, 0], r[:, :oh, 1]), r[:, 1:oh + 1, 0])
    r = t.reshape(tb, oh, w // 2, 2, c)
    return jnp.maximum(jnp.maximum(r[:, :, :ow, 0], r[:, :, :ow, 1]),
                       r[:, :, 1:ow + 1, 0])


def _conv3x3(x, w, b, n):
    tb, _, _, c = x.shape
    xe = _extend(_extend(x, 1, 3), 2, 3)
    a = jnp.concatenate(
        [xe[:, i:i + 16, j:j + 16, :]
         for i in range(3) for j in range(3)], axis=-1)
    a = a.reshape(tb * 16 * 16, 9 * c)
    y = jnp.dot(a, w, preferred_element_type=jnp.float32)
    y = jnp.maximum(y + b, 0.0)
    return y.reshape(tb, 16, 16, n)


def _features_body(x_ref, w0_ref, b0_ref, w1_ref, b1_ref, w2_ref, b2_ref,
                   w3_ref, b3_ref, w4_ref, b4_ref, o_ref):
    tb = x_ref.shape[0]

    x = _extend(_extend(x_ref[...], 1, 2), 2, 2)
    a = jnp.concatenate(
        [x[:, gi:gi + 56, gj:gj + 56, :]
         for gi in range(3) for gj in range(3)], axis=-1)
    a = a.reshape(tb * 56 * 56, 9 * 48)
    acc = jnp.dot(a, w0_ref[...], preferred_element_type=jnp.float32)
    y = jnp.maximum(acc + b0_ref[...], 0.0)
    y = y.reshape(tb, 56, 56, 48)
    y = _zpad2(_pool3x2(y, 27, 27).astype(jnp.bfloat16), 2)

    x = _extend(_extend(y, 1, 5), 2, 5)
    acc = None
    for i in range(5):
        a = jnp.concatenate(
            [x[:, i:i + 32, j:j + 32, :] for j in range(5)], axis=-1)
        a = a.reshape(tb * 32 * 32, 240)
        d = jnp.dot(a, w1_ref[i], preferred_element_type=jnp.float32)
        acc = d if acc is None else acc + d
    y = jnp.maximum(acc + b1_ref[...], 0.0)
    y = y.reshape(tb, 32, 32, 128)
    y = _zpad2(_pool3x2(y, 13, 13).astype(jnp.bfloat16), 1)

    y = _conv3x3(y, w2_ref[...], b2_ref[...], 192)
    y = _zpad2(y[:, :13, :13, :].astype(jnp.bfloat16), 1)
    y = _conv3x3(y, w3_ref[...], b3_ref[...], 192)
    y = _zpad2(y[:, :13, :13, :].astype(jnp.bfloat16), 1)
    y = _conv3x3(y, w4_ref[...], b4_ref[...], 128)
    o_ref[...] = _pool3x2(y, 6, 6).astype(o_ref.dtype)


def _conv_call(body, x, wbs, out_shape, out_dtype, tb):
    batch = x.shape[0]
    xs = x.shape[1:]
    os = out_shape
    wb_specs = [pl.BlockSpec(a.shape, functools.partial(
        lambda nd, i: (0,) * nd, a.ndim)) for a in wbs]
    return pl.pallas_call(
        body,
        out_shape=jax.ShapeDtypeStruct((batch,) + os, out_dtype),
        grid=(batch // tb,),
        in_specs=[pl.BlockSpec((tb,) + xs, lambda i: (i, 0, 0, 0))] + wb_specs,
        out_specs=pl.BlockSpec((tb,) + os, lambda i: (i, 0, 0, 0)),
        compiler_params=pltpu.CompilerParams(
            dimension_semantics=("parallel",),
            vmem_limit_bytes=_VMEM_LIMIT),
    )(x, *wbs)


def _fc_body(x_ref, w1_ref, b1_ref, w2_ref, b2_ref, w3_ref, b3_ref, o_ref):
    h = jnp.dot(x_ref[...], w1_ref[...], preferred_element_type=jnp.float32)
    h = jnp.maximum(h + b1_ref[...], 0.0).astype(jnp.bfloat16)
    h = jnp.dot(h, w2_ref[...], preferred_element_type=jnp.float32)
    h = jnp.maximum(h + b2_ref[...], 0.0).astype(jnp.bfloat16)
    o = jnp.dot(h, w3_ref[...], preferred_element_type=jnp.float32)
    o_ref[...] = o + b3_ref[...]


def _classifier(xf, w1, b1, w2, b2, w3, b3):
    batch, k1 = xf.shape
    nb = 2 if batch % 2 == 0 else 1
    tb = batch // nb
    nc = w3.shape[1]
    return pl.pallas_call(
        _fc_body,
        out_shape=jax.ShapeDtypeStruct((batch, nc), jnp.float32),
        grid=(nb,),
        in_specs=[
            pl.BlockSpec((tb, k1), lambda i: (i, 0)),
            pl.BlockSpec(w1.shape, lambda i: (0, 0)),
            pl.BlockSpec(b1.shape, lambda i: (0, 0)),
            pl.BlockSpec(w2.shape, lambda i: (0, 0)),
            pl.BlockSpec(b2.shape, lambda i: (0, 0)),
            pl.BlockSpec(w3.shape, lambda i: (0, 0)),
            pl.BlockSpec(b3.shape, lambda i: (0, 0)),
        ],
        out_specs=pl.BlockSpec((tb, nc), lambda i: (i, 0)),
        compiler_params=pltpu.CompilerParams(
            dimension_semantics=("parallel",),
            vmem_limit_bytes=_VMEM_LIMIT),
    )(xf, w1, b1, w2, b2, w3, b3)


def kernel(x, conv0_w, conv0_b, conv1_w, conv1_b, conv2_w, conv2_b,
           conv3_w, conv3_b, conv4_w, conv4_b,
           fc0_w, fc0_b, fc1_w, fc1_b, fc2_w, fc2_b):
    batch = x.shape[0]

    xb = x.astype(jnp.bfloat16)
    xp = jnp.pad(xb, ((0, 0), (0, 0), (2, 2), (2, 2)))
    xs = xp.reshape(batch, 3, 57, 4, 57, 4)
    xs = xs.transpose(0, 2, 4, 1, 3, 5).reshape(batch, 57, 57, 48)

    w4 = conv0_w.reshape(11, 11, 3, 48)
    w4 = jnp.pad(w4, ((0, 1), (0, 1), (0, 0), (0, 0)))
    w0 = w4.reshape(3, 4, 3, 4, 3, 48).transpose(0, 2, 4, 1, 3, 5)
    w0 = w0.reshape(432, 48)

    y = _conv_call(_features_body, xs,
                   [w0, conv0_b, conv1_w.reshape(5, 240, 128), conv1_b,
                    conv2_w, conv2_b, conv3_w, conv3_b, conv4_w, conv4_b],
                   (6, 6, 128), jnp.bfloat16, tb=4)

    xf = y.reshape(batch, 6 * 6 * 128)
    return _classifier(xf, fc0_w, fc0_b, fc1_w, fc1_b, fc2_w, fc2_b)


# speedup vs baseline: 1.4604x; 1.1118x over previous
import functools

import jax
import jax.numpy as jnp
from jax.experimental import pallas as pl
from jax.experimental.pallas import tpu as pltpu


_VMEM_LIMIT = 48 * 1024 * 1024


def _extend(t, axis, n):
    shp = list(t.shape)
    shp[axis] = n
    return jnp.concatenate([t, jnp.zeros(shp, t.dtype)], axis=axis)


def _zpad2(t, p):
    tb, h, w, c = t.shape
    zr = jnp.zeros((tb, p, w, c), t.dtype)
    t = jnp.concatenate([zr, t, zr], axis=1)
    zc = jnp.zeros((tb, h + 2 * p, p, c), t.dtype)
    return jnp.concatenate([zc, t, zc], axis=2)


def _pool3x2(t, oh, ow):
    tb, h, w, c = t.shape
    r = t.reshape(tb, h // 2, 2, w, c)
    t = jnp.maximum(jnp.maximum(r[:, :oh, 0], r[:, :oh, 1]), r[:, 1:oh + 1, 0])
    r = t.reshape(tb, oh, w // 2, 2, c)
    return jnp.maximum(jnp.maximum(r[:, :, :ow, 0], r[:, :, :ow, 1]),
                       r[:, :, 1:ow + 1, 0])


def _c0_body(x_ref, w_ref, b_ref, o_ref):
    tb = x_ref.shape[0]
    x = _extend(_extend(x_ref[...], 1, 2), 2, 2)
    a = jnp.concatenate(
        [x[:, gi:gi + 56, gj:gj + 56, :]
         for gi in range(3) for gj in range(3)], axis=-1)
    a = a.reshape(tb * 56 * 56, 9 * 48)
    acc = jnp.dot(a, w_ref[...], preferred_element_type=jnp.float32)
    y = jnp.maximum(acc + b_ref[...], 0.0)
    y = y.reshape(tb, 56, 56, 48)
    y = _pool3x2(y, 27, 27)
    o_ref[...] = _zpad2(y, 2).astype(o_ref.dtype)


def _c1_body(x_ref, w_ref, b_ref, o_ref):
    tb = x_ref.shape[0]
    x = _extend(_extend(x_ref[...], 1, 5), 2, 5)
    acc = None
    for i in range(5):
        a = jnp.concatenate(
            [x[:, i:i + 32, j:j + 32, :] for j in range(5)], axis=-1)
        a = a.reshape(tb * 32 * 32, 240)
        d = jnp.dot(a, w_ref[i], preferred_element_type=jnp.float32)
        acc = d if acc is None else acc + d
    y = jnp.maximum(acc + b_ref[...], 0.0)
    y = y.reshape(tb, 32, 32, 128)
    y = _pool3x2(y, 13, 13)
    o_ref[...] = _zpad2(y, 1).astype(o_ref.dtype)


def _conv3x3(x, w, b, n):
    tb, _, _, c = x.shape
    xe = _extend(_extend(x, 1, 3), 2, 3)
    a = jnp.concatenate(
        [xe[:, i:i + 16, j:j + 16, :]
         for i in range(3) for j in range(3)], axis=-1)
    a = a.reshape(tb * 16 * 16, 9 * c)
    y = jnp.dot(a, w, preferred_element_type=jnp.float32)
    y = jnp.maximum(y + b, 0.0)
    return y.reshape(tb, 16, 16, n)


def _c234_body(x_ref, w2_ref, b2_ref, w3_ref, b3_ref, w4_ref, b4_ref, o_ref):
    y = _conv3x3(x_ref[...], w2_ref[...], b2_ref[...], 192)
    y = _zpad2(y[:, :13, :13, :].astype(jnp.bfloat16), 1)
    y = _conv3x3(y, w3_ref[...], b3_ref[...], 192)
    y = _zpad2(y[:, :13, :13, :].astype(jnp.bfloat16), 1)
    y = _conv3x3(y, w4_ref[...], b4_ref[...], 128)
    o_ref[...] = _pool3x2(y, 6, 6).astype(o_ref.dtype)


def _conv_call(body, x, wbs, out_shape, out_dtype, tb):
    batch = x.shape[0]
    xs = x.shape[1:]
    os = out_shape
    wb_specs = [pl.BlockSpec(a.shape, functools.partial(
        lambda nd, i: (0,) * nd, a.ndim)) for a in wbs]
    return pl.pallas_call(
        body,
        out_shape=jax.ShapeDtypeStruct((batch,) + os, out_dtype),
        grid=(batch // tb,),
        in_specs=[pl.BlockSpec((tb,) + xs, lambda i: (i, 0, 0, 0))] + wb_specs,
        out_specs=pl.BlockSpec((tb,) + os, lambda i: (i, 0, 0, 0)),
        compiler_params=pltpu.CompilerParams(
            dimension_semantics=("parallel",),
            vmem_limit_bytes=_VMEM_LIMIT),
    )(x, *wbs)


def _fc_body(x_ref, w1_ref, b1_ref, w2_ref, b2_ref, w3_ref, b3_ref, o_ref):
    h = jnp.dot(x_ref[...], w1_ref[...], preferred_element_type=jnp.float32)
    h = jnp.maximum(h + b1_ref[...], 0.0).astype(jnp.bfloat16)
    h = jnp.dot(h, w2_ref[...], preferred_element_type=jnp.float32)
    h = jnp.maximum(h + b2_ref[...], 0.0).astype(jnp.bfloat16)
    o = jnp.dot(h, w3_ref[...], preferred_element_type=jnp.float32)
    o_ref[...] = o + b3_ref[...]


def _classifier(xf, w1, b1, w2, b2, w3, b3):
    batch, k1 = xf.shape
    nb = 2 if batch % 2 == 0 else 1
    tb = batch // nb
    nc = w3.shape[1]
    return pl.pallas_call(
        _fc_body,
        out_shape=jax.ShapeDtypeStruct((batch, nc), jnp.float32),
        grid=(nb,),
        in_specs=[
            pl.BlockSpec((tb, k1), lambda i: (i, 0)),
            pl.BlockSpec(w1.shape, lambda i: (0, 0)),
            pl.BlockSpec(b1.shape, lambda i: (0, 0)),
            pl.BlockSpec(w2.shape, lambda i: (0, 0)),
            pl.BlockSpec(b2.shape, lambda i: (0, 0)),
            pl.BlockSpec(w3.shape, lambda i: (0, 0)),
            pl.BlockSpec(b3.shape, lambda i: (0, 0)),
        ],
        out_specs=pl.BlockSpec((tb, nc), lambda i: (i, 0)),
        compiler_params=pltpu.CompilerParams(
            dimension_semantics=("parallel",),
            vmem_limit_bytes=_VMEM_LIMIT),
    )(xf, w1, b1, w2, b2, w3, b3)


def kernel(x, conv0_w, conv0_b, conv1_w, conv1_b, conv2_w, conv2_b,
           conv3_w, conv3_b, conv4_w, conv4_b,
           fc0_w, fc0_b, fc1_w, fc1_b, fc2_w, fc2_b):
    batch = x.shape[0]

    xb = x.astype(jnp.bfloat16)
    xp = jnp.pad(xb, ((0, 0), (0, 0), (2, 2), (2, 2)))
    x1 = xp.transpose(0, 2, 1, 3)
    xs = x1.reshape(batch, 57, 4, 3, 57, 4)
    xs = xs.transpose(0, 1, 4, 2, 3, 5).reshape(batch, 57, 57, 48)

    w4 = conv0_w.reshape(11, 11, 3, 48)
    w4 = jnp.pad(w4, ((0, 1), (0, 1), (0, 0), (0, 0)))
    w0 = w4.reshape(3, 4, 3, 4, 3, 48).transpose(0, 2, 1, 4, 3, 5)
    w0 = w0.reshape(432, 48)

    y = _conv_call(_c0_body, xs, [w0, conv0_b], (31, 31, 48),
                   jnp.bfloat16, tb=4)
    y = _conv_call(_c1_body, y, [conv1_w.reshape(5, 240, 128), conv1_b],
                   (15, 15, 128), jnp.bfloat16, tb=8)
    y = _conv_call(_c234_body, y,
                   [conv2_w, conv2_b, conv3_w, conv3_b, conv4_w, conv4_b],
                   (6, 6, 128), jnp.bfloat16, tb=8)

    xf = y.reshape(batch, 6 * 6 * 128)
    return _classifier(xf, fc0_w, fc0_b, fc1_w, fc1_b, fc2_w, fc2_b)


# speedup vs baseline: 1.4663x; 1.0041x over previous
import functools

import jax
import jax.numpy as jnp
from jax.experimental import pallas as pl
from jax.experimental.pallas import tpu as pltpu


_VMEM_LIMIT = 48 * 1024 * 1024


def _extend(t, axis, n):
    shp = list(t.shape)
    shp[axis] = n
    return jnp.concatenate([t, jnp.zeros(shp, t.dtype)], axis=axis)


def _zpad2(t, p):
    tb, h, w, c = t.shape
    zr = jnp.zeros((tb, p, w, c), t.dtype)
    t = jnp.concatenate([zr, t, zr], axis=1)
    zc = jnp.zeros((tb, h + 2 * p, p, c), t.dtype)
    return jnp.concatenate([zc, t, zc], axis=2)


def _pool3x2(t, oh, ow):
    tb, h, w, c = t.shape
    r = t.reshape(tb, h // 2, 2, w, c)
    t = jnp.maximum(jnp.maximum(r[:, :oh, 0], r[:, :oh, 1]), r[:, 1:oh + 1, 0])
    r = t.reshape(tb, oh, w // 2, 2, c)
    return jnp.maximum(jnp.maximum(r[:, :, :ow, 0], r[:, :, :ow, 1]),
                       r[:, :, 1:ow + 1, 0])


def _c0_body(x_ref, w_ref, b_ref, o_ref):
    tb = x_ref.shape[0]
    x = _extend(_extend(x_ref[...], 1, 2), 2, 2)
    a = jnp.concatenate(
        [x[:, gi:gi + 56, gj:gj + 56, :]
         for gi in range(3) for gj in range(3)], axis=-1)
    a = a.reshape(tb * 56 * 56, 9 * 48)
    acc = jnp.dot(a, w_ref[...], preferred_element_type=jnp.float32)
    y = jnp.maximum(acc + b_ref[...], 0.0)
    y = y.reshape(tb, 56, 56, 48)
    y = _pool3x2(y, 27, 27)
    o_ref[...] = _zpad2(y, 2).astype(o_ref.dtype)


def _c1_body(x_ref, w_ref, b_ref, o_ref):
    tb = x_ref.shape[0]
    x = _extend(_extend(x_ref[...], 1, 5), 2, 5)
    acc = None
    for i in range(5):
        a = jnp.concatenate(
            [x[:, i:i + 32, j:j + 32, :] for j in range(5)], axis=-1)
        a = a.reshape(tb * 32 * 32, 240)
        d = jnp.dot(a, w_ref[i], preferred_element_type=jnp.float32)
        acc = d if acc is None else acc + d
    y = jnp.maximum(acc + b_ref[...], 0.0)
    y = y.reshape(tb, 32, 32, 128)
    y = _pool3x2(y, 13, 13)
    o_ref[...] = _zpad2(y, 1).astype(o_ref.dtype)


def _conv3x3(x, w, b, n):
    tb, _, _, c = x.shape
    xe = _extend(_extend(x, 1, 3), 2, 3)
    a = jnp.concatenate(
        [xe[:, i:i + 16, j:j + 16, :]
         for i in range(3) for j in range(3)], axis=-1)
    a = a.reshape(tb * 16 * 16, 9 * c)
    y = jnp.dot(a, w, preferred_element_type=jnp.float32)
    y = jnp.maximum(y + b, 0.0)
    return y.reshape(tb, 16, 16, n)


def _c234_body(x_ref, w2_ref, b2_ref, w3_ref, b3_ref, w4_ref, b4_ref, o_ref):
    y = _conv3x3(x_ref[...], w2_ref[...], b2_ref[...], 192)
    y = _zpad2(y[:, :13, :13, :].astype(jnp.bfloat16), 1)
    y = _conv3x3(y, w3_ref[...], b3_ref[...], 192)
    y = _zpad2(y[:, :13, :13, :].astype(jnp.bfloat16), 1)
    y = _conv3x3(y, w4_ref[...], b4_ref[...], 128)
    o_ref[...] = _pool3x2(y, 6, 6).astype(o_ref.dtype)


def _conv_call(body, x, wbs, out_shape, out_dtype, tb):
    batch = x.shape[0]
    xs = x.shape[1:]
    os = out_shape
    wb_specs = [pl.BlockSpec(a.shape, functools.partial(
        lambda nd, i: (0,) * nd, a.ndim)) for a in wbs]
    return pl.pallas_call(
        body,
        out_shape=jax.ShapeDtypeStruct((batch,) + os, out_dtype),
        grid=(batch // tb,),
        in_specs=[pl.BlockSpec((tb,) + xs, lambda i: (i, 0, 0, 0))] + wb_specs,
        out_specs=pl.BlockSpec((tb,) + os, lambda i: (i, 0, 0, 0)),
        compiler_params=pltpu.CompilerParams(
            dimension_semantics=("parallel",),
            vmem_limit_bytes=_VMEM_LIMIT),
    )(x, *wbs)


def _fc_body(x_ref, w1_ref, b1_ref, w2_ref, b2_ref, w3_ref, b3_ref, o_ref):
    h = jnp.dot(x_ref[...], w1_ref[...], preferred_element_type=jnp.float32)
    h = jnp.maximum(h + b1_ref[...], 0.0).astype(jnp.bfloat16)
    h = jnp.dot(h, w2_ref[...], preferred_element_type=jnp.float32)
    h = jnp.maximum(h + b2_ref[...], 0.0).astype(jnp.bfloat16)
    o = jnp.dot(h, w3_ref[...], preferred_element_type=jnp.float32)
    o_ref[...] = o + b3_ref[...]


def _classifier(xf, w1, b1, w2, b2, w3, b3):
    batch, k1 = xf.shape
    nb = 2 if batch % 2 == 0 else 1
    tb = batch // nb
    nc = w3.shape[1]
    return pl.pallas_call(
        _fc_body,
        out_shape=jax.ShapeDtypeStruct((batch, nc), jnp.float32),
        grid=(nb,),
        in_specs=[
            pl.BlockSpec((tb, k1), lambda i: (i, 0)),
            pl.BlockSpec(w1.shape, lambda i: (0, 0)),
            pl.BlockSpec(b1.shape, lambda i: (0, 0)),
            pl.BlockSpec(w2.shape, lambda i: (0, 0)),
            pl.BlockSpec(b2.shape, lambda i: (0, 0)),
            pl.BlockSpec(w3.shape, lambda i: (0, 0)),
            pl.BlockSpec(b3.shape, lambda i: (0, 0)),
        ],
        out_specs=pl.BlockSpec((tb, nc), lambda i: (i, 0)),
        compiler_params=pltpu.CompilerParams(
            dimension_semantics=("parallel",),
            vmem_limit_bytes=_VMEM_LIMIT),
    )(xf, w1, b1, w2, b2, w3, b3)


def kernel(x, conv0_w, conv0_b, conv1_w, conv1_b, conv2_w, conv2_b,
           conv3_w, conv3_b, conv4_w, conv4_b,
           fc0_w, fc0_b, fc1_w, fc1_b, fc2_w, fc2_b):
    batch = x.shape[0]

    xb = x.astype(jnp.bfloat16)
    xp = jnp.pad(xb, ((0, 0), (0, 0), (2, 2), (2, 2)))
    x1 = xp.transpose(0, 2, 1, 3)
    xs = x1.reshape(batch, 57, 4, 3, 57, 4)
    xs = xs.transpose(0, 1, 4, 2, 3, 5).reshape(batch, 57, 57, 48)

    w4 = conv0_w.reshape(11, 11, 3, 48)
    w4 = jnp.pad(w4, ((0, 1), (0, 1), (0, 0), (0, 0)))
    w0 = w4.reshape(3, 4, 3, 4, 3, 48).transpose(0, 2, 1, 4, 3, 5)
    w0 = w0.reshape(432, 48)

    y = _conv_call(_c0_body, xs, [w0, conv0_b], (31, 31, 48),
                   jnp.bfloat16, tb=4)
    y = _conv_call(_c1_body, y, [conv1_w.reshape(5, 240, 128), conv1_b],
                   (15, 15, 128), jnp.bfloat16, tb=16)
    y = _conv_call(_c234_body, y,
                   [conv2_w, conv2_b, conv3_w, conv3_b, conv4_w, conv4_b],
                   (6, 6, 128), jnp.bfloat16, tb=16)

    xf = y.reshape(batch, 6 * 6 * 128)
    return _classifier(xf, fc0_w, fc0_b, fc1_w, fc1_b, fc2_w, fc2_b)
